# baseline (speedup 1.0000x reference)
import functools

import jax
import jax.numpy as jnp
from jax import lax
from jax.experimental import pallas as pl
from jax.experimental.pallas import tpu as pltpu

_LANES = 128
_TBLK = 128
_IBLK = 256
_SBLK = 256


def _shift_rows(m, k):
    return jnp.concatenate([jnp.zeros((k, m.shape[1]), m.dtype), m[:-k]], axis=0)


def _excl_cumsum_rows(m):
    s = m.shape[0]
    c = m
    k = 1
    while k < s:
        c = c + _shift_rows(c, k)
        k *= 2
    return c - m


def _excl_cumsum_lanes(v):
    n = v.shape[1]
    c = v
    k = 1
    while k < n:
        c = c + jnp.concatenate([jnp.zeros((1, k), v.dtype), c[:, :-k]], axis=1)
        k *= 2
    return c - v


def _routing_kernel(nrows, n_exp, x_ref, rw_ref, sgs_ref, tok_ref, w_ref,
                    off_ref, gate_ref):
    s = x_ref.shape[0]
    x = x_ref[...]
    rw = rw_ref[...]
    gate_ref[...] = lax.logistic(
        lax.dot_general(sgs_ref[...], x, (((1,), (1,)), ((), ())),
                        preferred_element_type=jnp.float32))
    logits = lax.dot_general(x, rw, (((1,), (1,)), ((), ())),
                             preferred_element_type=jnp.float32)
    m = jnp.max(logits, axis=1, keepdims=True)
    eu = jnp.exp(logits - m)
    z = jnp.sum(eu, axis=1, keepdims=True)
    p = eu / z
    iota = lax.broadcasted_iota(jnp.int32, (s, n_exp), 1)
    v1 = jnp.max(p, axis=1, keepdims=True)
    i1 = jnp.min(jnp.where(p == v1, iota, n_exp), axis=1, keepdims=True)
    oh1 = iota == i1
    pm = jnp.where(oh1, -jnp.inf, p)
    v2 = jnp.max(pm, axis=1, keepdims=True)
    i2 = jnp.min(jnp.where(pm == v2, iota, n_exp), axis=1, keepdims=True)
    oh2 = iota == i2
    s12 = v1 + v2
    w1 = v1 / s12
    w2 = v2 / s12
    f1 = oh1.astype(jnp.float32)
    f2 = oh2.astype(jnp.float32)
    cnt1 = jnp.sum(f1, axis=0, keepdims=True)
    rank1 = jnp.sum(_excl_cumsum_rows(f1) * f1, axis=1, keepdims=True)
    rank2 = jnp.sum((_excl_cumsum_rows(f2) + cnt1) * f2, axis=1, keepdims=True)
    counts = cnt1 + jnp.sum(f2, axis=0, keepdims=True)
    offs = _excl_cumsum_lanes(counts)
    off1 = jnp.sum(offs * f1, axis=1, keepdims=True)
    off2 = jnp.sum(offs * f2, axis=1, keepdims=True)
    pos1 = off1 + rank1
    pos2 = off2 + rank2
    tind = lax.broadcasted_iota(jnp.int32, (s, 1), 0)
    hi_vec = (tind // _LANES).astype(jnp.float32)
    lo_vec = (tind % _LANES).astype(jnp.float32)
    v1m = jnp.concatenate([hi_vec, lo_vec, w1], axis=1)
    v2m = jnp.concatenate([hi_vec, lo_vec, w2], axis=1)
    jrow = lax.broadcasted_iota(jnp.int32, (1, _LANES), 1).astype(jnp.float32)
    dn = (((0,), (0,)), ((), ()))
    for r in range(nrows):
        tgt = jrow + float(r * _LANES)
        c1 = (pos1 == tgt).astype(jnp.float32)
        c2 = (pos2 == tgt).astype(jnp.float32)
        m = (lax.dot_general(v1m, c1, dn, preferred_element_type=jnp.float32)
             + lax.dot_general(v2m, c2, dn, preferred_element_type=jnp.float32))
        tok_ref[r:r + 1, :] = m[0:1, :] * float(_LANES) + m[1:2, :]
        w_ref[r:r + 1, :] = m[2:3, :]
    pad = jnp.full((1, _LANES - n_exp), float(2 * s), jnp.float32)
    off_ref[...] = jnp.concatenate([offs, pad], axis=1)


def _expert_kernel(shared_rows, tok_sm, off_sm, w_sm, x_ref, gw_ref, uw_ref,
                   dw_ref, sgw_ref, suw_ref, sdw_ref, sgs_ref,
                   out_ref, chunk_ref):
    e = pl.program_id(0)

    @pl.when(e == 0)
    def _init():
        out_ref[...] = jnp.zeros_like(out_ref)

    start = off_sm[e]
    end = off_sm[e + 1]
    nch = (end - start + _TBLK - 1) // _TBLK
    dn_t = (((1,), (1,)), ((), ()))

    def chunk_body(c, carry):
        base = start + c * _TBLK
        for j in range(_TBLK):
            t = tok_sm[base + j]
            chunk_ref[j:j + 1, :] = x_ref[pl.ds(t, 1), :]
        xc = chunk_ref[...]
        xg = lax.dot_general(xc, gw_ref[0], dn_t, preferred_element_type=jnp.float32)
        xu = lax.dot_general(xc, uw_ref[0], dn_t, preferred_element_type=jnp.float32)
        a = (xg * lax.logistic(xg)) * xu
        chunk_ref[...] = lax.dot_general(a, dw_ref[0], dn_t,
                                         preferred_element_type=jnp.float32)
        for j in range(_TBLK):
            t = tok_sm[base + j]
            wj = jnp.where(base + j < end, w_sm[base + j], 0.0)
            out_ref[pl.ds(t, 1), :] = (out_ref[pl.ds(t, 1), :]
                                       + chunk_ref[j:j + 1, :] * wj)
        return carry

    lax.fori_loop(0, nch, chunk_body, 0)

    if shared_rows:
        row0 = pl.multiple_of(e * shared_rows, 8)
        xb = x_ref[pl.ds(row0, shared_rows), :].astype(jnp.bfloat16)
        g = lax.dot_general(xb, sgw_ref[...], dn_t,
                            preferred_element_type=jnp.float32)
        u = lax.dot_general(xb, suw_ref[...], dn_t,
                            preferred_element_type=jnp.float32)
        ash = ((g * lax.logistic(g)) * u).astype(jnp.bfloat16)
        sh = lax.dot_general(ash, sdw_ref[...], dn_t,
                             preferred_element_type=jnp.float32)
        gt = lax.transpose(sgs_ref[pl.ds(e, 1), :], (1, 0))
        out_ref[pl.ds(row0, shared_rows), :] = (
            out_ref[pl.ds(row0, shared_rows), :] + gt * sh)


def _shared_kernel(x_ref, eo_ref, sgw_ref, suw_ref, sdw_ref, sgs_ref, out_ref):
    x = x_ref[...]
    dn_t = (((1,), (1,)), ((), ()))
    g = lax.dot_general(x, sgw_ref[...], dn_t, preferred_element_type=jnp.float32)
    u = lax.dot_general(x, suw_ref[...], dn_t, preferred_element_type=jnp.float32)
    a = (g * lax.logistic(g)) * u
    sh = lax.dot_general(a, sdw_ref[...], dn_t, preferred_element_type=jnp.float32)
    gt = lax.logistic(lax.dot_general(x, sgs_ref[...], dn_t,
                                      preferred_element_type=jnp.float32))
    out_ref[...] = eo_ref[...] + gt * sh


def kernel(hidden_states, router_w, gate_w, up_w, down_w, shared_gate_w,
           shared_up_w, shared_down_w, shared_gate_scalar_w):
    b, seq, h = hidden_states.shape
    n_exp = router_w.shape[0]
    inter = gate_w.shape[1]
    s = b * seq
    x = hidden_states.reshape(s, h)
    n_assign = 2 * s
    nrows = n_assign // _LANES + 1

    tok_f, w_f, off_f, gate_vec = pl.pallas_call(
        functools.partial(_routing_kernel, nrows, n_exp),
        out_shape=[
            jax.ShapeDtypeStruct((nrows, _LANES), jnp.float32),
            jax.ShapeDtypeStruct((nrows, _LANES), jnp.float32),
            jax.ShapeDtypeStruct((1, _LANES), jnp.float32),
            jax.ShapeDtypeStruct((1, s), jnp.float32),
        ],
    )(x, router_w, shared_gate_scalar_w)

    tok_i = tok_f.reshape(-1).astype(jnp.int32)
    off_i = off_f.reshape(-1).astype(jnp.int32)
    w_flat = w_f.reshape(-1)

    shared_rows = s // n_exp if s % n_exp == 0 else 0
    gate_arg = (gate_vec.reshape(n_exp, shared_rows) if shared_rows
                else gate_vec)
    sgw16 = shared_gate_w.astype(jnp.bfloat16)
    suw16 = shared_up_w.astype(jnp.bfloat16)
    sdw16 = shared_down_w.astype(jnp.bfloat16)
    grid_spec = pltpu.PrefetchScalarGridSpec(
        num_scalar_prefetch=3,
        grid=(n_exp,),
        in_specs=[
            pl.BlockSpec((s, h), lambda e, *_: (0, 0)),
            pl.BlockSpec((1, inter, h), lambda e, *_: (e, 0, 0)),
            pl.BlockSpec((1, inter, h), lambda e, *_: (e, 0, 0)),
            pl.BlockSpec((1, h, inter), lambda e, *_: (e, 0, 0)),
            pl.BlockSpec(sgw16.shape, lambda e, *_: (0, 0)),
            pl.BlockSpec(suw16.shape, lambda e, *_: (0, 0)),
            pl.BlockSpec(sdw16.shape, lambda e, *_: (0, 0)),
            pl.BlockSpec(gate_arg.shape, lambda e, *_: (0, 0)),
        ],
        out_specs=pl.BlockSpec((s, h), lambda e, *_: (0, 0)),
        scratch_shapes=[
            pltpu.VMEM((_TBLK, h), jnp.float32),
        ],
    )
    out = pl.pallas_call(
        functools.partial(_expert_kernel, shared_rows),
        grid_spec=grid_spec,
        out_shape=jax.ShapeDtypeStruct((s, h), jnp.float32),
        compiler_params=pltpu.CompilerParams(
            vmem_limit_bytes=67_000_000,
        ),
    )(tok_i, off_i, w_flat, x, gate_w, up_w, down_w,
      sgw16, suw16, sdw16, gate_arg)

    if not shared_rows:
        sblk = _SBLK if s % _SBLK == 0 else s
        out = pl.pallas_call(
            _shared_kernel,
            grid=(s // sblk,),
            in_specs=[
                pl.BlockSpec((sblk, h), lambda t: (t, 0)),
                pl.BlockSpec((sblk, h), lambda t: (t, 0)),
                pl.BlockSpec(shared_gate_w.shape, lambda t: (0, 0)),
                pl.BlockSpec(shared_up_w.shape, lambda t: (0, 0)),
                pl.BlockSpec(shared_down_w.shape, lambda t: (0, 0)),
                pl.BlockSpec(shared_gate_scalar_w.shape, lambda t: (0, 0)),
            ],
            out_specs=pl.BlockSpec((sblk, h), lambda t: (t, 0)),
            out_shape=jax.ShapeDtypeStruct((s, h), jnp.float32),
        )(x, out, shared_gate_w, shared_up_w, shared_down_w,
          shared_gate_scalar_w)

    return out.reshape(b, seq, h)


# speedup vs baseline: 1.1119x; 1.1119x over previous
import functools

import jax
import jax.numpy as jnp
from jax import lax
from jax.experimental import pallas as pl
from jax.experimental.pallas import tpu as pltpu

_LANES = 128
_TBLK = 64
_IBLK = 256
_SBLK = 256


def _shift_rows(m, k):
    return jnp.concatenate([jnp.zeros((k, m.shape[1]), m.dtype), m[:-k]], axis=0)


def _excl_cumsum_rows(m):
    s = m.shape[0]
    c = m
    k = 1
    while k < s:
        c = c + _shift_rows(c, k)
        k *= 2
    return c - m


def _excl_cumsum_lanes(v):
    n = v.shape[1]
    c = v
    k = 1
    while k < n:
        c = c + jnp.concatenate([jnp.zeros((1, k), v.dtype), c[:, :-k]], axis=1)
        k *= 2
    return c - v


def _routing_kernel(nrows, n_exp, x_ref, rw_ref, tok_ref, w_ref, off_ref):
    s = x_ref.shape[0]
    x = x_ref[...]
    rw = rw_ref[...]
    logits = lax.dot_general(x, rw, (((1,), (1,)), ((), ())),
                             preferred_element_type=jnp.float32)
    m = jnp.max(logits, axis=1, keepdims=True)
    eu = jnp.exp(logits - m)
    z = jnp.sum(eu, axis=1, keepdims=True)
    p = eu / z
    iota = lax.broadcasted_iota(jnp.int32, (s, n_exp), 1)
    v1 = jnp.max(p, axis=1, keepdims=True)
    i1 = jnp.min(jnp.where(p == v1, iota, n_exp), axis=1, keepdims=True)
    oh1 = iota == i1
    pm = jnp.where(oh1, -jnp.inf, p)
    v2 = jnp.max(pm, axis=1, keepdims=True)
    i2 = jnp.min(jnp.where(pm == v2, iota, n_exp), axis=1, keepdims=True)
    oh2 = iota == i2
    s12 = v1 + v2
    w1 = v1 / s12
    w2 = v2 / s12
    f1 = oh1.astype(jnp.float32)
    f2 = oh2.astype(jnp.float32)
    cnt1 = jnp.sum(f1, axis=0, keepdims=True)
    rank1 = jnp.sum(_excl_cumsum_rows(f1) * f1, axis=1, keepdims=True)
    rank2 = jnp.sum((_excl_cumsum_rows(f2) + cnt1) * f2, axis=1, keepdims=True)
    counts = cnt1 + jnp.sum(f2, axis=0, keepdims=True)
    offs = _excl_cumsum_lanes(counts)
    off1 = jnp.sum(offs * f1, axis=1, keepdims=True)
    off2 = jnp.sum(offs * f2, axis=1, keepdims=True)
    pos1 = off1 + rank1
    pos2 = off2 + rank2
    tind = lax.broadcasted_iota(jnp.int32, (s, 1), 0)
    hi_vec = (tind // _LANES).astype(jnp.float32)
    lo_vec = (tind % _LANES).astype(jnp.float32)
    v1m = jnp.concatenate([hi_vec, lo_vec, w1], axis=1)
    v2m = jnp.concatenate([hi_vec, lo_vec, w2], axis=1)
    jrow = lax.broadcasted_iota(jnp.int32, (1, _LANES), 1).astype(jnp.float32)
    dn = (((0,), (0,)), ((), ()))
    for r in range(nrows):
        tgt = jrow + float(r * _LANES)
        c1 = (pos1 == tgt).astype(jnp.float32)
        c2 = (pos2 == tgt).astype(jnp.float32)
        m = (lax.dot_general(v1m, c1, dn, preferred_element_type=jnp.float32)
             + lax.dot_general(v2m, c2, dn, preferred_element_type=jnp.float32))
        tok_ref[r:r + 1, :] = m[0:1, :] * float(_LANES) + m[1:2, :]
        w_ref[r:r + 1, :] = m[2:3, :]
    pad = jnp.full((1, _LANES - n_exp), float(2 * s), jnp.float32)
    off_ref[...] = jnp.concatenate([offs, pad], axis=1)


def _expert_kernel(tok_sm, off_sm, w_sm, x_ref, gw_ref, uw_ref,
                   dw_ref, out_ref, chunk_ref):
    e = pl.program_id(0)

    @pl.when(e == 0)
    def _init():
        out_ref[...] = jnp.zeros_like(out_ref)

    start = off_sm[e]
    end = off_sm[e + 1]
    nch = (end - start + _TBLK - 1) // _TBLK
    dn_t = (((1,), (1,)), ((), ()))

    def chunk_body(c, carry):
        base = start + c * _TBLK
        for j in range(_TBLK):
            t = tok_sm[base + j]
            chunk_ref[j:j + 1, :] = x_ref[pl.ds(t, 1), :]
        xc = chunk_ref[...]
        xg = lax.dot_general(xc, gw_ref[0], dn_t, preferred_element_type=jnp.float32)
        xu = lax.dot_general(xc, uw_ref[0], dn_t, preferred_element_type=jnp.float32)
        a = (xg * lax.logistic(xg)) * xu
        chunk_ref[...] = lax.dot_general(a, dw_ref[0], dn_t,
                                         preferred_element_type=jnp.float32)
        for j in range(_TBLK):
            t = tok_sm[base + j]
            wj = jnp.where(base + j < end, w_sm[base + j], 0.0)
            out_ref[pl.ds(t, 1), :] = (out_ref[pl.ds(t, 1), :]
                                       + chunk_ref[j:j + 1, :] * wj)
        return carry

    lax.fori_loop(0, nch, chunk_body, 0)


def _shared_kernel(x_ref, eo_ref, sgw_ref, suw_ref, sdw_ref, sgs_ref, out_ref):
    x = x_ref[...]
    dn_t = (((1,), (1,)), ((), ()))
    g = lax.dot_general(x, sgw_ref[...], dn_t, preferred_element_type=jnp.float32)
    u = lax.dot_general(x, suw_ref[...], dn_t, preferred_element_type=jnp.float32)
    a = (g * lax.logistic(g)) * u
    sh = lax.dot_general(a, sdw_ref[...], dn_t, preferred_element_type=jnp.float32)
    gt = lax.logistic(lax.dot_general(x, sgs_ref[...], dn_t,
                                      preferred_element_type=jnp.float32))
    out_ref[...] = eo_ref[...] + gt * sh


def kernel(hidden_states, router_w, gate_w, up_w, down_w, shared_gate_w,
           shared_up_w, shared_down_w, shared_gate_scalar_w):
    b, seq, h = hidden_states.shape
    n_exp = router_w.shape[0]
    inter = gate_w.shape[1]
    s = b * seq
    x = hidden_states.reshape(s, h)
    n_assign = 2 * s
    nrows = n_assign // _LANES + 1

    tok_f, w_f, off_f = pl.pallas_call(
        functools.partial(_routing_kernel, nrows, n_exp),
        out_shape=[
            jax.ShapeDtypeStruct((nrows, _LANES), jnp.float32),
            jax.ShapeDtypeStruct((nrows, _LANES), jnp.float32),
            jax.ShapeDtypeStruct((1, _LANES), jnp.float32),
        ],
    )(x, router_w)

    tok_i = tok_f.reshape(-1).astype(jnp.int32)
    off_i = off_f.reshape(-1).astype(jnp.int32)
    w_flat = w_f.reshape(-1)

    grid_spec = pltpu.PrefetchScalarGridSpec(
        num_scalar_prefetch=3,
        grid=(n_exp,),
        in_specs=[
            pl.BlockSpec((s, h), lambda e, *_: (0, 0)),
            pl.BlockSpec((1, inter, h), lambda e, *_: (e, 0, 0)),
            pl.BlockSpec((1, inter, h), lambda e, *_: (e, 0, 0)),
            pl.BlockSpec((1, h, inter), lambda e, *_: (e, 0, 0)),
        ],
        out_specs=pl.BlockSpec((s, h), lambda e, *_: (0, 0)),
        scratch_shapes=[
            pltpu.VMEM((_TBLK, h), jnp.float32),
        ],
    )
    expert_out = pl.pallas_call(
        _expert_kernel,
        grid_spec=grid_spec,
        out_shape=jax.ShapeDtypeStruct((s, h), jnp.float32),
        compiler_params=pltpu.CompilerParams(
            vmem_limit_bytes=67_000_000,
        ),
    )(tok_i, off_i, w_flat, x, gate_w, up_w, down_w)

    sblk = _SBLK if s % _SBLK == 0 else s
    out = pl.pallas_call(
        _shared_kernel,
        grid=(s // sblk,),
        in_specs=[
            pl.BlockSpec((sblk, h), lambda t: (t, 0)),
            pl.BlockSpec((sblk, h), lambda t: (t, 0)),
            pl.BlockSpec(shared_gate_w.shape, lambda t: (0, 0)),
            pl.BlockSpec(shared_up_w.shape, lambda t: (0, 0)),
            pl.BlockSpec(shared_down_w.shape, lambda t: (0, 0)),
            pl.BlockSpec(shared_gate_scalar_w.shape, lambda t: (0, 0)),
        ],
        out_specs=pl.BlockSpec((sblk, h), lambda t: (t, 0)),
        out_shape=jax.ShapeDtypeStruct((s, h), jnp.float32),
    )(x, expert_out, shared_gate_w, shared_up_w, shared_down_w,
      shared_gate_scalar_w)

    return out.reshape(b, seq, h)


# speedup vs baseline: 1.2000x; 1.0792x over previous
import functools

import jax
import jax.numpy as jnp
from jax import lax
from jax.experimental import pallas as pl
from jax.experimental.pallas import tpu as pltpu

_LANES = 128
_TBLK = 128
_IBLK = 256
_SBLK = 256


def _shift_rows(m, k):
    return jnp.concatenate([jnp.zeros((k, m.shape[1]), m.dtype), m[:-k]], axis=0)


def _excl_cumsum_rows(m):
    s = m.shape[0]
    c = m
    k = 1
    while k < s:
        c = c + _shift_rows(c, k)
        k *= 2
    return c - m


def _excl_cumsum_lanes(v):
    n = v.shape[1]
    c = v
    k = 1
    while k < n:
        c = c + jnp.concatenate([jnp.zeros((1, k), v.dtype), c[:, :-k]], axis=1)
        k *= 2
    return c - v


def _routing_kernel(nrows, n_exp, x_ref, rw_ref, tok_ref, w_ref, off_ref):
    s = x_ref.shape[0]
    x = x_ref[...]
    rw = rw_ref[...]
    logits = lax.dot_general(x, rw, (((1,), (1,)), ((), ())),
                             preferred_element_type=jnp.float32)
    m = jnp.max(logits, axis=1, keepdims=True)
    eu = jnp.exp(logits - m)
    z = jnp.sum(eu, axis=1, keepdims=True)
    p = eu / z
    iota = lax.broadcasted_iota(jnp.int32, (s, n_exp), 1)
    v1 = jnp.max(p, axis=1, keepdims=True)
    i1 = jnp.min(jnp.where(p == v1, iota, n_exp), axis=1, keepdims=True)
    oh1 = iota == i1
    pm = jnp.where(oh1, -jnp.inf, p)
    v2 = jnp.max(pm, axis=1, keepdims=True)
    i2 = jnp.min(jnp.where(pm == v2, iota, n_exp), axis=1, keepdims=True)
    oh2 = iota == i2
    s12 = v1 + v2
    w1 = v1 / s12
    w2 = v2 / s12
    f1 = oh1.astype(jnp.float32)
    f2 = oh2.astype(jnp.float32)
    cnt1 = jnp.sum(f1, axis=0, keepdims=True)
    rank1 = jnp.sum(_excl_cumsum_rows(f1) * f1, axis=1, keepdims=True)
    rank2 = jnp.sum((_excl_cumsum_rows(f2) + cnt1) * f2, axis=1, keepdims=True)
    counts = cnt1 + jnp.sum(f2, axis=0, keepdims=True)
    offs = _excl_cumsum_lanes(counts)
    off1 = jnp.sum(offs * f1, axis=1, keepdims=True)
    off2 = jnp.sum(offs * f2, axis=1, keepdims=True)
    pos1 = off1 + rank1
    pos2 = off2 + rank2
    tind = lax.broadcasted_iota(jnp.int32, (s, 1), 0)
    hi_vec = (tind // _LANES).astype(jnp.float32)
    lo_vec = (tind % _LANES).astype(jnp.float32)
    v1m = jnp.concatenate([hi_vec, lo_vec, w1], axis=1)
    v2m = jnp.concatenate([hi_vec, lo_vec, w2], axis=1)
    jrow = lax.broadcasted_iota(jnp.int32, (1, _LANES), 1).astype(jnp.float32)
    dn = (((0,), (0,)), ((), ()))
    for r in range(nrows):
        tgt = jrow + float(r * _LANES)
        c1 = (pos1 == tgt).astype(jnp.float32)
        c2 = (pos2 == tgt).astype(jnp.float32)
        m = (lax.dot_general(v1m, c1, dn, preferred_element_type=jnp.float32)
             + lax.dot_general(v2m, c2, dn, preferred_element_type=jnp.float32))
        tok_ref[r:r + 1, :] = m[0:1, :] * float(_LANES) + m[1:2, :]
        w_ref[r:r + 1, :] = m[2:3, :]
    pad = jnp.full((1, _LANES - n_exp), float(2 * s), jnp.float32)
    off_ref[...] = jnp.concatenate([offs, pad], axis=1)


def _expert_kernel(tok_sm, off_sm, w_sm, x_ref, gw_ref, uw_ref,
                   dw_ref, out_ref, chunk_ref):
    e = pl.program_id(0)

    @pl.when(e == 0)
    def _init():
        out_ref[...] = jnp.zeros_like(out_ref)

    start = off_sm[e]
    end = off_sm[e + 1]
    nch = (end - start + _TBLK - 1) // _TBLK
    dn_t = (((1,), (1,)), ((), ()))

    def chunk_body(c, carry):
        base = start + c * _TBLK
        for j in range(_TBLK):
            t = tok_sm[base + j]
            chunk_ref[j:j + 1, :] = x_ref[pl.ds(t, 1), :]
        xc = chunk_ref[...]
        xg = lax.dot_general(xc, gw_ref[0], dn_t, preferred_element_type=jnp.float32)
        xu = lax.dot_general(xc, uw_ref[0], dn_t, preferred_element_type=jnp.float32)
        a = (xg * lax.logistic(xg)) * xu
        chunk_ref[...] = lax.dot_general(a, dw_ref[0], dn_t,
                                         preferred_element_type=jnp.float32)
        for j in range(_TBLK):
            t = tok_sm[base + j]
            wj = jnp.where(base + j < end, w_sm[base + j], 0.0)
            out_ref[pl.ds(t, 1), :] = (out_ref[pl.ds(t, 1), :]
                                       + chunk_ref[j:j + 1, :] * wj)
        return carry

    lax.fori_loop(0, nch, chunk_body, 0)


def _shared_kernel(x_ref, eo_ref, sgw_ref, suw_ref, sdw_ref, sgs_ref, out_ref):
    x = x_ref[...]
    dn_t = (((1,), (1,)), ((), ()))
    g = lax.dot_general(x, sgw_ref[...], dn_t, preferred_element_type=jnp.float32)
    u = lax.dot_general(x, suw_ref[...], dn_t, preferred_element_type=jnp.float32)
    a = (g * lax.logistic(g)) * u
    sh = lax.dot_general(a, sdw_ref[...], dn_t, preferred_element_type=jnp.float32)
    gt = lax.logistic(lax.dot_general(x, sgs_ref[...], dn_t,
                                      preferred_element_type=jnp.float32))
    out_ref[...] = eo_ref[...] + gt * sh


def kernel(hidden_states, router_w, gate_w, up_w, down_w, shared_gate_w,
           shared_up_w, shared_down_w, shared_gate_scalar_w):
    b, seq, h = hidden_states.shape
    n_exp = router_w.shape[0]
    inter = gate_w.shape[1]
    s = b * seq
    x = hidden_states.reshape(s, h)
    n_assign = 2 * s
    nrows = n_assign // _LANES + 1

    tok_f, w_f, off_f = pl.pallas_call(
        functools.partial(_routing_kernel, nrows, n_exp),
        out_shape=[
            jax.ShapeDtypeStruct((nrows, _LANES), jnp.float32),
            jax.ShapeDtypeStruct((nrows, _LANES), jnp.float32),
            jax.ShapeDtypeStruct((1, _LANES), jnp.float32),
        ],
    )(x, router_w)

    tok_i = tok_f.reshape(-1).astype(jnp.int32)
    off_i = off_f.reshape(-1).astype(jnp.int32)
    w_flat = w_f.reshape(-1)

    grid_spec = pltpu.PrefetchScalarGridSpec(
        num_scalar_prefetch=3,
        grid=(n_exp,),
        in_specs=[
            pl.BlockSpec((s, h), lambda e, *_: (0, 0)),
            pl.BlockSpec((1, inter, h), lambda e, *_: (e, 0, 0)),
            pl.BlockSpec((1, inter, h), lambda e, *_: (e, 0, 0)),
            pl.BlockSpec((1, h, inter), lambda e, *_: (e, 0, 0)),
        ],
        out_specs=pl.BlockSpec((s, h), lambda e, *_: (0, 0)),
        scratch_shapes=[
            pltpu.VMEM((_TBLK, h), jnp.float32),
        ],
    )
    expert_out = pl.pallas_call(
        _expert_kernel,
        grid_spec=grid_spec,
        out_shape=jax.ShapeDtypeStruct((s, h), jnp.float32),
        compiler_params=pltpu.CompilerParams(
            vmem_limit_bytes=67_000_000,
        ),
    )(tok_i, off_i, w_flat, x, gate_w, up_w, down_w)

    sblk = _SBLK if s % _SBLK == 0 else s
    out = pl.pallas_call(
        _shared_kernel,
        grid=(s // sblk,),
        in_specs=[
            pl.BlockSpec((sblk, h), lambda t: (t, 0)),
            pl.BlockSpec((sblk, h), lambda t: (t, 0)),
            pl.BlockSpec(shared_gate_w.shape, lambda t: (0, 0)),
            pl.BlockSpec(shared_up_w.shape, lambda t: (0, 0)),
            pl.BlockSpec(shared_down_w.shape, lambda t: (0, 0)),
            pl.BlockSpec(shared_gate_scalar_w.shape, lambda t: (0, 0)),
        ],
        out_specs=pl.BlockSpec((sblk, h), lambda t: (t, 0)),
        out_shape=jax.ShapeDtypeStruct((s, h), jnp.float32),
    )(x, expert_out, shared_gate_w, shared_up_w, shared_down_w,
      shared_gate_scalar_w)

    return out.reshape(b, seq, h)
